# baseline (speedup 1.0000x reference)
import functools

import jax
import jax.numpy as jnp
from jax import lax
from jax.experimental import pallas as pl
from jax.experimental.pallas import tpu as pltpu
from jax.experimental.pallas import tpu_sc as plsc

B = 16384
ED = 64
NC, NS = 2, 16
NW = NC * NS
BPW = B // NW

_sc_mesh = plsc.VectorSubcoreMesh(core_axis_name="c", subcore_axis_name="s")


@functools.partial(
    pl.kernel,
    mesh=_sc_mesh,
    out_type=[
        jax.ShapeDtypeStruct((B, ED), jnp.float32),
        jax.ShapeDtypeStruct((B, ED), jnp.float32),
    ],
    scratch_types=[
        pltpu.VMEM((BPW,), jnp.int32),
        pltpu.VMEM((BPW,), jnp.int32),
        pltpu.VMEM((BPW, ED), jnp.float32),
        pltpu.VMEM((BPW, ED), jnp.float32),
        pltpu.SemaphoreType.DMA,
        pltpu.SemaphoreType.DMA,
    ],
    compiler_params=pltpu.CompilerParams(use_tc_tiling_on_sc=False),
)
def _gather_sc(uid_hbm, iid_hbm, utab_hbm, itab_hbm, uout_hbm, iout_hbm,
               uidx_v, iidx_v, urows_v, irows_v, usem, isem):
    wid = lax.axis_index("s") * NC + lax.axis_index("c")
    base = wid * BPW
    pltpu.sync_copy(uid_hbm.at[pl.ds(base, BPW)], uidx_v)
    pltpu.sync_copy(iid_hbm.at[pl.ds(base, BPW)], iidx_v)
    ucp = pltpu.async_copy(utab_hbm.at[uidx_v], urows_v, usem)
    icp = pltpu.async_copy(itab_hbm.at[iidx_v], irows_v, isem)
    ucp.wait()
    pltpu.sync_copy(urows_v, uout_hbm.at[pl.ds(base, BPW)])
    icp.wait()
    pltpu.sync_copy(irows_v, iout_hbm.at[pl.ds(base, BPW)])


BLK = 2048


def _mlp_body(feat_ref, ue_ref, ie_ref, w1_ref, b1_ref, w2_ref, b2_ref,
              w3u_ref, w3i_ref, w3f_ref, b3_ref, w4_ref, b4_ref,
              w5_ref, b5_ref, out_ref):
    h = jnp.maximum(
        jnp.dot(feat_ref[...], w1_ref[...], preferred_element_type=jnp.float32)
        + b1_ref[...], 0.0)
    f = jnp.maximum(
        jnp.dot(h, w2_ref[...], preferred_element_type=jnp.float32)
        + b2_ref[...], 0.0)
    y = (jnp.dot(ue_ref[...], w3u_ref[...], preferred_element_type=jnp.float32)
         + jnp.dot(ie_ref[...], w3i_ref[...], preferred_element_type=jnp.float32)
         + jnp.dot(f, w3f_ref[...], preferred_element_type=jnp.float32)
         + b3_ref[...])
    y = jnp.maximum(y, 0.0)
    y = jnp.maximum(
        jnp.dot(y, w4_ref[...], preferred_element_type=jnp.float32)
        + b4_ref[...], 0.0)
    z = (jnp.dot(y, w5_ref[...], preferred_element_type=jnp.float32)
         + b5_ref[...])
    out_ref[...] = 1.0 / (1.0 + jnp.exp(-z))


def _mlp_tc(features, ue, ie, W1, b1, W2, b2, W3u, W3i, W3f, b3, W4, b4, W5, b5):
    nblk = B // BLK
    row_spec = lambda w: pl.BlockSpec((BLK, w), lambda i: (i, 0))
    full = lambda a: pl.BlockSpec(a.shape, lambda i: (0,) * a.ndim)
    return pl.pallas_call(
        _mlp_body,
        grid=(nblk,),
        in_specs=[
            row_spec(features.shape[1]),
            row_spec(ED),
            row_spec(ED),
            full(W1), full(b1), full(W2), full(b2),
            full(W3u), full(W3i), full(W3f), full(b3),
            full(W4), full(b4), full(W5), full(b5),
        ],
        out_specs=pl.BlockSpec((BLK, 1), lambda i: (i, 0)),
        out_shape=jax.ShapeDtypeStruct((B, 1), jnp.float32),
    )(features, ue, ie, W1, b1, W2, b2, W3u, W3i, W3f, b3, W4, b4, W5, b5)


def kernel(user_ids, item_ids, features, user_table, item_table,
           W1, b1, W2, b2, W3, b3, W4, b4, W5, b5):
    ue, ie = _gather_sc(user_ids.astype(jnp.int32), item_ids.astype(jnp.int32),
                        user_table, item_table)
    out = _mlp_tc(
        features, ue, ie,
        W1, b1.reshape(1, -1), W2, b2.reshape(1, -1),
        W3[:ED], W3[ED:2 * ED], W3[2 * ED:], b3.reshape(1, -1),
        W4, b4.reshape(1, -1), W5, b5.reshape(1, -1))
    return out.reshape(B)
